# baseline (speedup 1.0000x reference)
import jax
import jax.numpy as jnp
from jax.experimental import pallas as pl


def _add_kernel(x_ref, e_ref, o_ref):
    o_ref[...] = x_ref[...] + e_ref[...]


def kernel(x, emb_weight):
    B, N, T, D = x.shape
    x2 = x.reshape(B, N, T * D)
    emb_t = jnp.tile(emb_weight, (1, T))

    out = pl.pallas_call(
        _add_kernel,
        grid=(B,),
        in_specs=[
            pl.BlockSpec((1, N, T * D), lambda b: (b, 0, 0)),
            pl.BlockSpec((N, T * D), lambda b: (0, 0)),
        ],
        out_specs=pl.BlockSpec((1, N, T * D), lambda b: (b, 0, 0)),
        out_shape=jax.ShapeDtypeStruct((B, N, T * D), x.dtype),
    )(x2, emb_t)
    return out.reshape(B, N, T, D)


# speedup vs baseline: 1.4537x; 1.4537x over previous
import jax
import jax.numpy as jnp
from jax.experimental import pallas as pl
from jax.experimental.pallas import tpu as pltpu

_NB = 128


def _add_kernel(x_ref, e_ref, o_ref):
    o_ref[...] = x_ref[...] + e_ref[...][None, :, None, :]


def kernel(x, emb_weight):
    B, N, T, D = x.shape
    grid = (B, pl.cdiv(N, _NB))
    return pl.pallas_call(
        _add_kernel,
        grid=grid,
        in_specs=[
            pl.BlockSpec((1, _NB, T, D), lambda b, j: (b, j, 0, 0)),
            pl.BlockSpec((_NB, D), lambda b, j: (j, 0)),
        ],
        out_specs=pl.BlockSpec((1, _NB, T, D), lambda b, j: (b, j, 0, 0)),
        out_shape=jax.ShapeDtypeStruct((B, N, T, D), x.dtype),
        compiler_params=pltpu.CompilerParams(
            dimension_semantics=("parallel", "parallel"),
        ),
    )(x, emb_weight)


# speedup vs baseline: 1.7636x; 1.2132x over previous
import jax
import jax.numpy as jnp
from jax.experimental import pallas as pl
from jax.experimental.pallas import tpu as pltpu

_NB = 500
_BUFS = 2


def _add_kernel(x_ref, e_ref, o_ref):
    o_ref[...] = x_ref[...] + e_ref[...][None, :, None, :]


def kernel(x, emb_weight):
    B, N, T, D = x.shape
    return pl.pallas_call(
        _add_kernel,
        grid=(B, pl.cdiv(N, _NB)),
        in_specs=[
            pl.BlockSpec((1, _NB, T, D), lambda b, j: (b, j, 0, 0),
                         pipeline_mode=pl.Buffered(buffer_count=_BUFS)),
            pl.BlockSpec((_NB, D), lambda b, j: (j, 0)),
        ],
        out_specs=pl.BlockSpec((1, _NB, T, D), lambda b, j: (b, j, 0, 0),
                               pipeline_mode=pl.Buffered(buffer_count=_BUFS)),
        out_shape=jax.ShapeDtypeStruct((B, N, T, D), x.dtype),
        compiler_params=pltpu.CompilerParams(
            dimension_semantics=("parallel", "parallel"),
        ),
    )(x, emb_weight)
